# Initial kernel scaffold; baseline (speedup 1.0000x reference)
import functools

import jax
import jax.numpy as jnp
from jax import lax
from jax.experimental import pallas as pl
from jax.experimental.pallas import tpu as pltpu
from jax.experimental.pallas import tpu_sc as plsc

N_NODES = 10000
N_EDGES = 20000
N_PAIRS = 160000
D = 512

NC = 2
NS = 16
L = 16

PPT = N_PAIRS // NS
NVEC = PPT // L
SELCAP = ((PPT + 127) // 128) * 128 + 16
NBMAX = SELCAP // 128
BATCH = 128

B_CSC = 3584
B_PASSES = 3
E_PAD = B_PASSES * NC * B_CSC
B_RPT = B_CSC // NS

C_CSC = 2560
C_PASSES = 2
V_PAD = C_PASSES * NC * C_CSC
C_RPT = C_CSC // NS


def _fill_i32(ref, n, value):
    def body(i, _):
        ref[pl.ds(i * L, L)] = jnp.full((L,), value, jnp.int32)
        return 0
    lax.fori_loop(0, n // L, body, 0)


def _zero_rows(ref, nrows):
    z = jnp.zeros((L,), jnp.float32)
    def body(r, _):
        for k in range(D // L):
            ref[r, pl.ds(k * L, L)] = z
        return 0
    lax.fori_loop(0, nrows, body, 0)


def _compact(sidx_ref, gidx_ref, base, csize, sel_g_ref, sel_l_flat_ref):
    def body(i, off):
        sv = sidx_ref[pl.ds(i * L, L)]
        gv = gidx_ref[pl.ds(i * L, L)]
        lv = sv - base
        mask = (lv >= 0) & (lv < csize)
        plsc.store_compressed(sel_g_ref.at[pl.ds(off, L)], gv, mask=mask)
        plsc.store_compressed(sel_l_flat_ref.at[pl.ds(off, L)], lv, mask=mask)
        return off + jnp.sum(mask.astype(jnp.int32))
    return lax.fori_loop(0, NVEC, body, jnp.int32(0))


def _to_rows(flat_ref, rows_ref, nb):
    def body(j, _):
        for k in range(BATCH // L):
            rows_ref[j, pl.ds(k * L, L)] = flat_ref[pl.ds(j * BATCH + k * L, L)]
        return 0
    lax.fori_loop(0, nb, body, 0)


def _edge_agg_body(xp_hbm, v_hbm, e_hbm, xe_hbm,
                   pv, pe, selg, self_, sel3, rows, ones, cntl,
                   accum, cnts, sem):
    c = lax.axis_index("c")
    t = lax.axis_index("s")

    pltpu.sync_copy(v_hbm.at[pl.ds(t * PPT, PPT)], pv)
    pltpu.sync_copy(e_hbm.at[pl.ds(t * PPT, PPT)], pe)

    def ones_body(j, _):
        ones[j] = jnp.full((L,), 1.0, jnp.float32)
        return 0
    lax.fori_loop(0, BATCH, ones_body, 0)

    for p in range(B_PASSES):
        base = p * NC * B_CSC + c * B_CSC

        _zero_rows(rows, BATCH)
        def czero(r, _):
            cntl[r] = jnp.zeros((L,), jnp.float32)
            return 0
        lax.fori_loop(0, B_RPT, czero, 0)
        for b in range(2):
            pltpu.sync_copy(rows.at[pl.ds(0, B_RPT // 2)],
                            accum.at[pl.ds(t * B_RPT + b * (B_RPT // 2), B_RPT // 2)])
        pltpu.sync_copy(cntl, cnts.at[pl.ds(t * B_RPT, B_RPT)])
        @pl.when(t == 0)
        def _():
            pltpu.sync_copy(rows.at[pl.ds(0, 1)], accum.at[pl.ds(B_CSC, 1)])
            pltpu.sync_copy(cntl.at[pl.ds(0, 1)], cnts.at[pl.ds(B_CSC, 1)])
        plsc.subcore_barrier()

        _fill_i32(selg, SELCAP, 0)
        _fill_i32(self_, SELCAP, B_CSC)
        nsel = _compact(pe, pv, base, B_CSC, selg, self_)
        nb = (nsel + BATCH - 1) // BATCH
        _to_rows(self_, sel3, nb)

        def gs_body(j, _):
            pltpu.async_copy(xp_hbm.at[selg.at[pl.ds(j * BATCH, BATCH)]], rows, sem).wait()
            pltpu.sync_copy(rows, accum.at[sel3.at[j]], add=True)
            pltpu.sync_copy(ones, cnts.at[sel3.at[j]], add=True)
            return 0
        lax.fori_loop(0, nb, gs_body, 0)
        plsc.subcore_barrier()

        pltpu.sync_copy(cnts.at[pl.ds(t * B_RPT, B_RPT)], cntl)
        hw = B_RPT // 2
        for b in range(2):
            pltpu.sync_copy(accum.at[pl.ds(t * B_RPT + b * hw, hw)],
                            rows.at[pl.ds(0, hw)])
            def scale_body(r, _):
                cr = cntl[b * hw + r]
                s = 1.0 / jnp.maximum(cr, 1.0)
                for k in range(D // L):
                    rows[r, pl.ds(k * L, L)] = rows[r, pl.ds(k * L, L)] * s
                return 0
            lax.fori_loop(0, hw, scale_body, 0)
            pltpu.sync_copy(rows.at[pl.ds(0, hw)],
                            xe_hbm.at[pl.ds(base + t * B_RPT + b * hw, hw)])


_edge_agg = functools.partial(
    pl.kernel,
    out_type=jax.ShapeDtypeStruct((E_PAD, D), jnp.float32),
    mesh=plsc.VectorSubcoreMesh(core_axis_name="c", subcore_axis_name="s",
                                num_cores=NC, num_subcores=NS),
    scratch_types=[
        pltpu.VMEM((PPT,), jnp.int32),
        pltpu.VMEM((PPT,), jnp.int32),
        pltpu.VMEM((SELCAP,), jnp.int32),
        pltpu.VMEM((SELCAP,), jnp.int32),
        pltpu.VMEM((NBMAX, BATCH), jnp.int32),
        pltpu.VMEM((BATCH, D), jnp.float32),
        pltpu.VMEM((BATCH, L), jnp.float32),
        pltpu.VMEM((B_RPT, L), jnp.float32),
        pltpu.VMEM_SHARED((B_CSC + 1, D), jnp.float32),
        pltpu.VMEM_SHARED((B_CSC + 1, L), jnp.float32),
        pltpu.SemaphoreType.DMA,
    ],
)(_edge_agg_body)


def _vertex_agg_body(xe_hbm, v_hbm, e_hbm, xv_hbm,
                     pv, pe, selg, self_, sel3, rows, accum, sem):
    c = lax.axis_index("c")
    t = lax.axis_index("s")

    pltpu.sync_copy(v_hbm.at[pl.ds(t * PPT, PPT)], pv)
    pltpu.sync_copy(e_hbm.at[pl.ds(t * PPT, PPT)], pe)

    for p in range(C_PASSES):
        base = p * NC * C_CSC + c * C_CSC

        _zero_rows(rows, BATCH)
        for b in range(2):
            pltpu.sync_copy(rows.at[pl.ds(0, C_RPT // 2)],
                            accum.at[pl.ds(t * C_RPT + b * (C_RPT // 2), C_RPT // 2)])
        @pl.when(t == 0)
        def _():
            pltpu.sync_copy(rows.at[pl.ds(0, 1)], accum.at[pl.ds(C_CSC, 1)])
        plsc.subcore_barrier()

        _fill_i32(selg, SELCAP, 0)
        _fill_i32(self_, SELCAP, C_CSC)
        nsel = _compact(pv, pe, base, C_CSC, selg, self_)
        nb = (nsel + BATCH - 1) // BATCH
        _to_rows(self_, sel3, nb)

        def gs_body(j, _):
            pltpu.async_copy(xe_hbm.at[selg.at[pl.ds(j * BATCH, BATCH)]], rows, sem).wait()
            pltpu.sync_copy(rows, accum.at[sel3.at[j]], add=True)
            return 0
        lax.fori_loop(0, nb, gs_body, 0)
        plsc.subcore_barrier()

        pltpu.sync_copy(accum.at[pl.ds(t * C_RPT, C_RPT)],
                        xv_hbm.at[pl.ds(base + t * C_RPT, C_RPT)])


_vertex_agg = functools.partial(
    pl.kernel,
    out_type=jax.ShapeDtypeStruct((V_PAD, D), jnp.float32),
    mesh=plsc.VectorSubcoreMesh(core_axis_name="c", subcore_axis_name="s",
                                num_cores=NC, num_subcores=NS),
    scratch_types=[
        pltpu.VMEM((PPT,), jnp.int32),
        pltpu.VMEM((PPT,), jnp.int32),
        pltpu.VMEM((SELCAP,), jnp.int32),
        pltpu.VMEM((SELCAP,), jnp.int32),
        pltpu.VMEM((NBMAX, BATCH), jnp.int32),
        pltpu.VMEM((BATCH, D), jnp.float32),
        pltpu.VMEM_SHARED((C_CSC + 1, D), jnp.float32),
        pltpu.SemaphoreType.DMA,
    ],
)(_vertex_agg_body)


def _mm_body(x_ref, w_ref, o_ref):
    o_ref[...] = jnp.dot(x_ref[...], w_ref[...],
                         preferred_element_type=jnp.float32)


def _matmul(x, w):
    m, k = x.shape
    _, n = w.shape
    bm = 1000
    return pl.pallas_call(
        _mm_body,
        grid=(m // bm,),
        in_specs=[pl.BlockSpec((bm, k), lambda i: (i, 0)),
                  pl.BlockSpec((k, n), lambda i: (0, 0))],
        out_specs=pl.BlockSpec((bm, n), lambda i: (i, 0)),
        out_shape=jax.ShapeDtypeStruct((m, n), jnp.float32),
    )(x, w)


def _fin_body(a_ref, b_ref, o_ref):
    s = a_ref[...] + b_ref[...]
    ss = jnp.sum(s * s, axis=1, keepdims=True)
    scale = jnp.where(ss > 0, lax.rsqrt(ss), 0.0)
    o_ref[...] = s * scale


def _finalize(xp, xv):
    m, n = xp.shape
    bm = 1000
    return pl.pallas_call(
        _fin_body,
        grid=(m // bm,),
        in_specs=[pl.BlockSpec((bm, n), lambda i: (i, 0)),
                  pl.BlockSpec((bm, n), lambda i: (i, 0))],
        out_specs=pl.BlockSpec((bm, n), lambda i: (i, 0)),
        out_shape=jax.ShapeDtypeStruct((m, n), jnp.float32),
    )(xp, xv)


def kernel(X, vertex, edges, W):
    xp = _matmul(X, W)
    xe = _edge_agg(xp, vertex, edges)
    xv = _vertex_agg(xe, vertex, edges)
    return _finalize(xp, xv[:N_NODES])


# speedup vs baseline: 2.3765x; 2.3765x over previous
import functools

import jax
import jax.numpy as jnp
from jax import lax
from jax.experimental import pallas as pl
from jax.experimental.pallas import tpu as pltpu
from jax.experimental.pallas import tpu_sc as plsc

N_NODES = 10000
N_EDGES = 20000
N_PAIRS = 160000
D = 512

NC = 2
NS = 16
L = 16

NW = NC * NS
PPT = N_PAIRS // NS
CH = 2000
NCH = PPT // CH
CHV = CH // L
BATCH = 32
SELCAP = ((PPT + BATCH - 1) // BATCH) * BATCH + L
NBMAX = SELCAP // BATCH + 1

B_CSC = 2048
B_PASSES = 5
E_PAD = B_PASSES * NC * B_CSC
C_CSC = 2688
C_PASSES = 2
V_PAD = C_PASSES * NC * C_CSC


def _seg_agg_body(csc, passes, with_counts,
                  table_hbm, g_hbm, s_hbm, *rest):
    if with_counts:
        (sums_hbm, cnts_hbm, gvb, svb, gsel, ssel3, rows, hist,
         accum, sem, sem2) = rest
    else:
        (sums_hbm, gvb, svb, gsel, ssel3, rows, accum, sem, sem2) = rest
        cnts_hbm = hist = None

    c = lax.axis_index("c")
    t = lax.axis_index("s")
    rpt = csc // NS

    if with_counts:
        zv = jnp.zeros((L,), jnp.float32)
        def hzero(i, _):
            hist[pl.ds(i * L, L)] = zv
            return 0
        lax.fori_loop(0, E_PAD // L, hzero, 0)
        onev = jnp.full((L,), 1.0, jnp.float32)
        @pl.when(c == 0)
        def _():
            for ch in range(NCH):
                pltpu.sync_copy(s_hbm.at[pl.ds(t * PPT + ch * CH, CH)], svb)
                def hbody(i, _):
                    plsc.addupdate_scatter(hist, [svb[pl.ds(i * L, L)]], onev)
                    return 0
                lax.fori_loop(0, CHV, hbody, 0)
        wid = t * NC + c
        pltpu.sync_copy(hist, cnts_hbm.at[pl.ds(wid * E_PAD, E_PAD)])

    for p in range(passes):
        base = (p * NC + c) * csc

        z = jnp.zeros((L,), jnp.float32)
        def zrow(r, _):
            for k in range(D // L):
                rows[r, pl.ds(k * L, L)] = z
            return 0
        lax.fori_loop(0, BATCH, zrow, 0)
        done = 0
        while done < rpt:
            n = min(BATCH, rpt - done)
            pltpu.sync_copy(rows.at[pl.ds(0, n)],
                            accum.at[pl.ds(t * rpt + done, n)])
            done += n
        @pl.when(t == 0)
        def _():
            pltpu.sync_copy(rows.at[pl.ds(0, 8)], accum.at[pl.ds(csc, 8)])
        plsc.subcore_barrier()

        zi = jnp.zeros((L,), jnp.int32)
        def gfill(i, _):
            gsel[pl.ds(i * L, L)] = zi
            return 0
        lax.fori_loop(0, SELCAP // L, gfill, 0)
        dmp = jnp.full((L,), csc, jnp.int32)
        def sfill(j, _):
            for k in range(BATCH // L):
                ssel3[j, pl.ds(k * L, L)] = dmp
            return 0
        lax.fori_loop(0, NBMAX, sfill, 0)

        bvec = jnp.full((L,), base, jnp.int32)
        cvec = jnp.full((L,), csc, jnp.int32)
        def cbody(i, off):
            sv = svb[pl.ds(i * L, L)]
            gv = gvb[pl.ds(i * L, L)]
            lv = sv - bvec
            mask = (lv >= 0) & (lv < cvec)
            mi = mask.astype(jnp.int32)
            cs = plsc.cumsum(mi)
            pos = jnp.full((L,), off, jnp.int32) + cs - mi
            plsc.store_scatter(gsel, [pos], gv, mask=mask)
            plsc.store_scatter(
                ssel3,
                [lax.shift_right_logical(pos, 5), pos & (BATCH - 1)],
                lv, mask=mask)
            return off + cs[L - 1]
        off = jnp.int32(0)
        for ch in range(NCH):
            pltpu.sync_copy(g_hbm.at[pl.ds(t * PPT + ch * CH, CH)], gvb)
            pltpu.sync_copy(s_hbm.at[pl.ds(t * PPT + ch * CH, CH)], svb)
            off = lax.fori_loop(0, CHV, cbody, off)
        nb = (off + BATCH - 1) // BATCH

        def gs(j, _):
            pltpu.async_copy(table_hbm.at[gsel.at[pl.ds(j * BATCH, BATCH)]],
                             rows, sem).wait()
            pltpu.async_copy(rows, accum.at[ssel3.at[j]], sem2,
                             add=True).wait()
            return 0
        lax.fori_loop(0, nb, gs, 0)
        plsc.subcore_barrier()

        pltpu.sync_copy(accum.at[pl.ds(t * rpt, rpt)],
                        sums_hbm.at[pl.ds(base + t * rpt, rpt)])


_sc_mesh = dict(core_axis_name="c", subcore_axis_name="s",
                num_cores=NC, num_subcores=NS)
_sc_params = dict(needs_layout_passes=False, use_tc_tiling_on_sc=False)


def _edge_agg(xp, vertex, edges):
    body = functools.partial(_seg_agg_body, B_CSC, B_PASSES, True)
    return pl.kernel(
        body,
        out_type=(jax.ShapeDtypeStruct((E_PAD, D), jnp.float32),
                  jax.ShapeDtypeStruct((NW * E_PAD,), jnp.float32)),
        mesh=plsc.VectorSubcoreMesh(**_sc_mesh),
        compiler_params=pltpu.CompilerParams(**_sc_params),
        scratch_types=[
            pltpu.VMEM((CH,), jnp.int32),
            pltpu.VMEM((CH,), jnp.int32),
            pltpu.VMEM((SELCAP,), jnp.int32),
            pltpu.VMEM((NBMAX, BATCH), jnp.int32),
            pltpu.VMEM((BATCH, D), jnp.float32),
            pltpu.VMEM((E_PAD,), jnp.float32),
            pltpu.VMEM_SHARED((B_CSC + 8, D), jnp.float32),
            pltpu.SemaphoreType.DMA,
            pltpu.SemaphoreType.DMA,
        ],
    )(xp, vertex, edges)


def _vertex_agg(xe, vertex, edges):
    body = functools.partial(_seg_agg_body, C_CSC, C_PASSES, False)
    return pl.kernel(
        body,
        out_type=jax.ShapeDtypeStruct((V_PAD, D), jnp.float32),
        mesh=plsc.VectorSubcoreMesh(**_sc_mesh),
        compiler_params=pltpu.CompilerParams(**_sc_params),
        scratch_types=[
            pltpu.VMEM((CH,), jnp.int32),
            pltpu.VMEM((CH,), jnp.int32),
            pltpu.VMEM((SELCAP,), jnp.int32),
            pltpu.VMEM((NBMAX, BATCH), jnp.int32),
            pltpu.VMEM((BATCH, D), jnp.float32),
            pltpu.VMEM_SHARED((C_CSC + 8, D), jnp.float32),
            pltpu.SemaphoreType.DMA,
            pltpu.SemaphoreType.DMA,
        ],
    )(xe, edges, vertex)


def _mm_body(x_ref, w_ref, o_ref):
    o_ref[...] = jnp.dot(x_ref[...], w_ref[...],
                         preferred_element_type=jnp.float32)


def _matmul(x, w):
    m, k = x.shape
    _, n = w.shape
    bm = 1000
    return pl.pallas_call(
        _mm_body,
        grid=(m // bm,),
        in_specs=[pl.BlockSpec((bm, k), lambda i: (i, 0)),
                  pl.BlockSpec((k, n), lambda i: (0, 0))],
        out_specs=pl.BlockSpec((bm, n), lambda i: (i, 0)),
        out_shape=jax.ShapeDtypeStruct((m, n), jnp.float32),
    )(x, w)


def _csum_body(c_ref, o_ref):
    o_ref[...] = jnp.sum(c_ref[...], axis=0)


def _count_combine(cnts):
    c3 = cnts.reshape(NW, E_PAD, 1)
    bm = 1024
    return pl.pallas_call(
        _csum_body,
        grid=(E_PAD // bm,),
        in_specs=[pl.BlockSpec((NW, bm, 1), lambda i: (0, i, 0))],
        out_specs=pl.BlockSpec((bm, 1), lambda i: (i, 0)),
        out_shape=jax.ShapeDtypeStruct((E_PAD, 1), jnp.float32),
    )(c3)


BM_E = 1024


def _mean_body(s_ref, c_ref, o_ref):
    o_ref[...] = s_ref[...] / jnp.maximum(c_ref[...], 1.0)


def _edge_mean(sums, cnt):
    return pl.pallas_call(
        _mean_body,
        grid=(E_PAD // BM_E,),
        in_specs=[pl.BlockSpec((BM_E, D), lambda i: (i, 0)),
                  pl.BlockSpec((BM_E, 1), lambda i: (i, 0))],
        out_specs=pl.BlockSpec((BM_E, D), lambda i: (i, 0)),
        out_shape=jax.ShapeDtypeStruct((E_PAD, D), jnp.float32),
    )(sums, cnt)


def _fin_body(xp_ref, v_ref, o_ref):
    s = xp_ref[...] + v_ref[...]
    ss = jnp.sum(s * s, axis=1, keepdims=True)
    scale = jnp.where(ss > 0, lax.rsqrt(ss), 0.0)
    o_ref[...] = s * scale


def _finalize(xp, xv):
    bm = 1000
    return pl.pallas_call(
        _fin_body,
        grid=(N_NODES // bm,),
        in_specs=[pl.BlockSpec((bm, D), lambda i: (i, 0)),
                  pl.BlockSpec((bm, D), lambda i: (i, 0))],
        out_specs=pl.BlockSpec((bm, D), lambda i: (i, 0)),
        out_shape=jax.ShapeDtypeStruct((N_NODES, D), jnp.float32),
    )(xp, xv)


def kernel(X, vertex, edges, W):
    xp = _matmul(X, W)
    sums, cnts = _edge_agg(xp, vertex, edges)
    cnt = _count_combine(cnts)
    xe = _edge_mean(sums, cnt)
    xv = _vertex_agg(xe, vertex, edges)
    return _finalize(xp, xv[:N_NODES])
